# Initial kernel scaffold; baseline (speedup 1.0000x reference)
import jax
import jax.numpy as jnp
from jax.experimental import pallas as pl


def kernel(node_feats, W, b, a):
    raise NotImplementedError("write your pallas kernel here")


# speedup vs baseline: 1.9463x; 1.9463x over previous
import jax
import jax.numpy as jnp
from jax.experimental import pallas as pl
from jax.experimental.pallas import tpu as pltpu

_ALPHA = 0.3
_NEG = -1e9


def _gat_kernel(x_ref, w_ref, b_ref, al_ref, ar_ref, out_ref, *, num_heads, c_head):
    x = x_ref[0]
    nrm = jnp.sqrt(jnp.sum(x * x, axis=1, keepdims=True))
    n = x / jnp.maximum(nrm, 1e-12)
    sim = jax.lax.dot_general(n, n, (((1,), (1,)), ((), ())),
                              preferred_element_type=jnp.float32)
    mask = sim > 0.0
    feats = jax.lax.dot_general(x, w_ref[...], (((1,), (1,)), ((), ())),
                                preferred_element_type=jnp.float32)
    feats = feats + b_ref[...][None, :]
    lcol = jnp.dot(feats, al_ref[...],
                   preferred_element_type=jnp.float32)
    rrow = jax.lax.dot_general(ar_ref[...], feats, (((0,), (1,)), ((), ())),
                               preferred_element_type=jnp.float32)
    for h in range(num_heads):
        lg = lcol[:, h:h + 1] + rrow[h:h + 1, :]
        lg = jnp.where(lg >= 0, lg, _ALPHA * lg)
        lg = jnp.where(mask, lg, _NEG)
        m = jnp.max(lg, axis=1, keepdims=True)
        e = jnp.where(mask, jnp.exp(lg - m), 0.0)
        p = e / jnp.sum(e, axis=1, keepdims=True)
        f_h = feats[:, h * c_head:(h + 1) * c_head]
        out_ref[0, :, h * c_head:(h + 1) * c_head] = jnp.dot(
            p, f_h, preferred_element_type=jnp.float32)


def kernel(node_feats, W, b, a):
    B, N, C = node_feats.shape
    H = a.shape[0]
    c_head = a.shape[1] // 2
    O = H * c_head
    eye = jnp.eye(H, dtype=a.dtype)
    Al = (a[:, :c_head, None] * eye[:, None, :]).reshape(O, H)
    Ar = (a[:, c_head:, None] * eye[:, None, :]).reshape(O, H)

    grid = (B,)
    out = pl.pallas_call(
        lambda *refs: _gat_kernel(*refs, num_heads=H, c_head=c_head),
        grid=grid,
        in_specs=[
            pl.BlockSpec((1, N, C), lambda i: (i, 0, 0)),
            pl.BlockSpec((O, C), lambda i: (0, 0)),
            pl.BlockSpec((O,), lambda i: (0,)),
            pl.BlockSpec((O, H), lambda i: (0, 0)),
            pl.BlockSpec((O, H), lambda i: (0, 0)),
        ],
        out_specs=pl.BlockSpec((1, N, O), lambda i: (i, 0, 0)),
        out_shape=jax.ShapeDtypeStruct((B, N, O), jnp.float32),
        compiler_params=pltpu.CompilerParams(
            dimension_semantics=("parallel",)),
    )(node_feats, W, b, Al, Ar)
    return out


# speedup vs baseline: 2.3588x; 1.2119x over previous
import jax
import jax.numpy as jnp
from jax.experimental import pallas as pl
from jax.experimental.pallas import tpu as pltpu

_ALPHA = 0.3


def _gat_kernel(x_ref, w_ref, b_ref, al_ref, ar_ref, out_ref, *, num_heads, c_head):
    x = x_ref[0]
    nrm = jnp.sqrt(jnp.sum(x * x, axis=1, keepdims=True))
    n = x / jnp.maximum(nrm, 1e-12)
    sim = jax.lax.dot_general(n, n, (((1,), (1,)), ((), ())),
                              preferred_element_type=jnp.float32)
    mask_f = jnp.where(sim > 0.0, 1.0, 0.0)
    feats = jax.lax.dot_general(x, w_ref[...], (((1,), (1,)), ((), ())),
                                preferred_element_type=jnp.float32)
    feats = feats + b_ref[...][None, :]
    lcol = jnp.dot(feats, al_ref[...],
                   preferred_element_type=jnp.float32)
    rcol = jnp.dot(feats, ar_ref[...],
                   preferred_element_type=jnp.float32)
    rrow = jax.lax.dot_general(ar_ref[...], feats, (((0,), (1,)), ((), ())),
                               preferred_element_type=jnp.float32)
    for h in range(num_heads):
        l_h = lcol[:, h:h + 1]
        r_h = rcol[:, h:h + 1]
        ge = rrow[h:h + 1, :] >= (0.0 - l_h)
        A = jnp.where(ge, mask_f, 0.0)
        Bm = mask_f - A
        mr = jnp.max(r_h)
        er = jnp.exp(r_h - mr)
        ear = jnp.exp(_ALPHA * (r_h - mr))
        f_h = feats[:, h * c_head:(h + 1) * c_head]
        G1 = jnp.concatenate([er * f_h, er], axis=1)
        G2 = jnp.concatenate([ear * f_h, ear], axis=1)
        AG = jnp.dot(A, G1, preferred_element_type=jnp.float32)
        BG = jnp.dot(Bm, G2, preferred_element_type=jnp.float32)
        t1 = l_h + mr
        t2 = _ALPHA * t1
        c = jnp.maximum(t1, t2)
        w1 = jnp.exp(t1 - c)
        w2 = jnp.exp(t2 - c)
        num = w1 * AG[:, :c_head] + w2 * BG[:, :c_head]
        s = w1 * AG[:, c_head:c_head + 1] + w2 * BG[:, c_head:c_head + 1]
        out_ref[0, :, h * c_head:(h + 1) * c_head] = num / s


def kernel(node_feats, W, b, a):
    B, N, C = node_feats.shape
    H = a.shape[0]
    c_head = a.shape[1] // 2
    O = H * c_head
    eye = jnp.eye(H, dtype=a.dtype)
    Al = (a[:, :c_head, None] * eye[:, None, :]).reshape(O, H)
    Ar = (a[:, c_head:, None] * eye[:, None, :]).reshape(O, H)

    grid = (B,)
    out = pl.pallas_call(
        lambda *refs: _gat_kernel(*refs, num_heads=H, c_head=c_head),
        grid=grid,
        in_specs=[
            pl.BlockSpec((1, N, C), lambda i: (i, 0, 0)),
            pl.BlockSpec((O, C), lambda i: (0, 0)),
            pl.BlockSpec((O,), lambda i: (0,)),
            pl.BlockSpec((O, H), lambda i: (0, 0)),
            pl.BlockSpec((O, H), lambda i: (0, 0)),
        ],
        out_specs=pl.BlockSpec((1, N, O), lambda i: (i, 0, 0)),
        out_shape=jax.ShapeDtypeStruct((B, N, O), jnp.float32),
        compiler_params=pltpu.CompilerParams(
            dimension_semantics=("parallel",)),
    )(node_feats, W, b, Al, Ar)
    return out
